# baseline (speedup 1.0000x reference)
import jax
import jax.numpy as jnp
from jax.experimental import pallas as pl
from jax.experimental.pallas import tpu as pltpu

_N = 4096
_NFEAT = 128
_F0, _F1, _F2 = 64, 68, 72
_SUMF = _F0 + _F1 + _F2
_H4 = _F0 * 2 + 4
_H5 = _F0 * 2
_BM = 512
_NB = _N // _BM


def _dot(a, b):
    return jnp.dot(a, b, preferred_element_type=jnp.float32)


def _cast_body(adj1_ref, adj2_ref, out1_ref, out2_ref):
    p = pl.program_id(0)

    @pl.when(p == 0)
    def _():
        out1_ref[...] = adj1_ref[...].astype(jnp.bfloat16)

    @pl.when(p == 1)
    def _():
        out2_ref[...] = adj2_ref[...].astype(jnp.bfloat16)


def _cast_pass(adj1, adj2):
    last = _NB - 1
    return pl.pallas_call(
        _cast_body,
        grid=(2, _NB),
        in_specs=[
            pl.BlockSpec((_BM, _N), lambda p, i: (i * (1 - p) + last * p, 0)),
            pl.BlockSpec((_BM, _N), lambda p, i: (i * p, 0)),
        ],
        out_specs=[
            pl.BlockSpec((_BM, _N), lambda p, i: (i * (1 - p) + last * p, 0)),
            pl.BlockSpec((_BM, _N), lambda p, i: (i * p, 0)),
        ],
        out_shape=[
            jax.ShapeDtypeStruct((_N, _N), jnp.bfloat16),
            jax.ShapeDtypeStruct((_N, _N), jnp.bfloat16),
        ],
        compiler_params=pltpu.CompilerParams(
            dimension_semantics=("arbitrary", "arbitrary")),
    )(adj1, adj2)


def _body(adj1_ref, adj2_ref, x_ref, y_ref, wl_ref, bl_ref, w4_ref, b4_ref,
          w5_ref, b5_ref, wmt_ref, bm_ref,
          xlr_ref, ylr_ref, xfin_ref, yfin_ref,
          xfiv_ref, xmlp_ref, yfiv_ref, ymlp_ref,
          lr_f32, lr_bf16):
    p = pl.program_id(0)
    i = pl.program_id(1)
    bf16 = jnp.bfloat16

    def layer1(adj_ref, feat_ref, lr_out_ref):
        ax = _dot(adj_ref[...], feat_ref[...])
        s = jax.nn.sigmoid(_dot(ax.astype(bf16), wl_ref[...]) + bl_ref[...])
        lane = jax.lax.broadcasted_iota(jnp.int32, (1, _SUMF), 1)
        sec_mask = jnp.logical_and(lane >= _F0, lane < _F0 + _F1)
        msec = jnp.sum(jnp.where(sec_mask, s, 0.0), axis=1,
                       keepdims=True) * (1.0 / _F1)
        lrb = s * jnp.where(lane < _F0 + _F1, 1.0, msec)
        lr_out_ref[...] = lrb
        lr_f32[pl.ds(i * _BM, _BM), :] = lrb
        lr_bf16[pl.ds(i * _BM, _BM), :] = lrb.astype(bf16)

    def layer2(adj_ref, final_ref, fiv_ref, mlp_ref):
        alr = _dot(adj_ref[...], lr_bf16[...])
        alrb = alr.astype(bf16)
        fou = _dot(alrb, w4_ref[...]) + b4_ref[...]
        fiv = _dot(alrb, w5_ref[...]) + b5_ref[...]
        m = _dot(fiv.astype(bf16), wmt_ref[...]) + bm_ref[...]
        m = jnp.where(m >= 0, m, 0.01 * m)
        f3 = (m + fou) * 0.5
        lrb = lr_f32[pl.ds(i * _BM, _BM), :]
        low = jnp.mean(lrb, axis=1, keepdims=True) * lrb + lrb
        final_ref[...] = jnp.concatenate([low, f3], axis=1)
        fiv_ref[...] = fiv
        mlp_ref[...] = m

    @pl.when(p == 0)
    def _():
        layer1(adj1_ref, x_ref, xlr_ref)

    @pl.when(p == 1)
    def _():
        layer2(adj1_ref, xfin_ref, xfiv_ref, xmlp_ref)

    @pl.when(p == 2)
    def _():
        layer1(adj2_ref, y_ref, ylr_ref)

    @pl.when(p == 3)
    def _():
        layer2(adj2_ref, yfin_ref, yfiv_ref, ymlp_ref)


def _const(shape):
    return pl.BlockSpec(shape, lambda p, i: tuple(0 for _ in shape))


def kernel(x, adj1, y, adj2, W1, b1, W2, b2, W3, b3, W4, b4, W5, b5, Wm, bm):
    f32 = jnp.float32
    bf16 = jnp.bfloat16
    wl = jnp.concatenate([W1, W2, W3], axis=1).astype(bf16)
    bl = jnp.concatenate([b1, b2, b3]).reshape(1, _SUMF)
    b4r = b4.reshape(1, _H4)
    b5r = b5.reshape(1, _H5)
    bmr = bm.reshape(1, _H4)
    xb = x.astype(bf16)
    yb = y.astype(bf16)
    w4b = W4.astype(bf16)
    w5b = W5.astype(bf16)
    wmtb = Wm.T.astype(bf16)

    adj1b, adj2b = _cast_pass(adj1, adj2)

    last = _NB - 1

    def adj1_map(p, i):
        c = p // 2
        return (i * (1 - c) + last * c, 0)

    def adj2_map(p, i):
        c = p // 2
        return (i * c, 0)

    def xlr_map(p, i):
        a = (p + 3) // 4
        return (i * (1 - a) + last * a, 0)

    def xtail_map(p, i):
        a = (p + 3) // 4
        b = p // 2
        return (i * (a - b) + last * b, 0)

    def ylr_map(p, i):
        c = p // 2
        d = p // 3
        return (i * (c - d) + last * d, 0)

    def ytail_map(p, i):
        d = p // 3
        return (i * d, 0)

    x_lr, y_lr, x_final, y_final, x_fiv, x_mlp, y_fiv, y_mlp = pl.pallas_call(
        _body,
        grid=(4, _NB),
        in_specs=[
            pl.BlockSpec((_BM, _N), adj1_map),
            pl.BlockSpec((_BM, _N), adj2_map),
            _const((_N, _NFEAT)),
            _const((_N, _NFEAT)),
            _const((_NFEAT, _SUMF)),
            _const((1, _SUMF)),
            _const((_SUMF, _H4)),
            _const((1, _H4)),
            _const((_SUMF, _H5)),
            _const((1, _H5)),
            _const((_H5, _H4)),
            _const((1, _H4)),
        ],
        out_specs=[
            pl.BlockSpec((_BM, _SUMF), xlr_map),
            pl.BlockSpec((_BM, _SUMF), ylr_map),
            pl.BlockSpec((_BM, _SUMF + _H4), xtail_map),
            pl.BlockSpec((_BM, _SUMF + _H4), ytail_map),
            pl.BlockSpec((_BM, _H5), xtail_map),
            pl.BlockSpec((_BM, _H4), xtail_map),
            pl.BlockSpec((_BM, _H5), ytail_map),
            pl.BlockSpec((_BM, _H4), ytail_map),
        ],
        out_shape=[
            jax.ShapeDtypeStruct((_N, _SUMF), f32),
            jax.ShapeDtypeStruct((_N, _SUMF), f32),
            jax.ShapeDtypeStruct((_N, _SUMF + _H4), f32),
            jax.ShapeDtypeStruct((_N, _SUMF + _H4), f32),
            jax.ShapeDtypeStruct((_N, _H5), f32),
            jax.ShapeDtypeStruct((_N, _H4), f32),
            jax.ShapeDtypeStruct((_N, _H5), f32),
            jax.ShapeDtypeStruct((_N, _H4), f32),
        ],
        scratch_shapes=[
            pltpu.VMEM((_N, _SUMF), f32),
            pltpu.VMEM((_N, _SUMF), jnp.bfloat16),
        ],
        compiler_params=pltpu.CompilerParams(
            dimension_semantics=("arbitrary", "arbitrary")),
    )(adj1b, adj2b, xb, yb, wl, bl, w4b, b4r, w5b, b5r, wmtb, bmr)
    return (x_lr, y_lr, x_final, y_final, x_fiv, x_mlp, y_fiv, y_mlp)


# speedup vs baseline: 1.0574x; 1.0574x over previous
import jax
import jax.numpy as jnp
from jax.experimental import pallas as pl
from jax.experimental.pallas import tpu as pltpu

_N = 4096
_NFEAT = 128
_F0, _F1, _F2 = 64, 68, 72
_SUMF = _F0 + _F1 + _F2
_H4 = _F0 * 2 + 4
_H5 = _F0 * 2
_BM = 512
_NB = _N // _BM
_BMC = 1024
_NBC = _N // _BMC


def _dot(a, b):
    return jnp.dot(a, b, preferred_element_type=jnp.float32)


def _cast_body(adj1_ref, adj2_ref, out1_ref, out2_ref):
    p = pl.program_id(0)

    @pl.when(p == 0)
    def _():
        out1_ref[...] = adj1_ref[...].astype(jnp.bfloat16)

    @pl.when(p == 1)
    def _():
        out2_ref[...] = adj2_ref[...].astype(jnp.bfloat16)


def _cast_pass(adj1, adj2):
    last = _NB - 1
    return pl.pallas_call(
        _cast_body,
        grid=(2, _NB),
        in_specs=[
            pl.BlockSpec((_BM, _N), lambda p, i: (i * (1 - p) + last * p, 0)),
            pl.BlockSpec((_BM, _N), lambda p, i: (i * p, 0)),
        ],
        out_specs=[
            pl.BlockSpec((_BM, _N), lambda p, i: (i * (1 - p) + last * p, 0)),
            pl.BlockSpec((_BM, _N), lambda p, i: (i * p, 0)),
        ],
        out_shape=[
            jax.ShapeDtypeStruct((_N, _N), jnp.bfloat16),
            jax.ShapeDtypeStruct((_N, _N), jnp.bfloat16),
        ],
        compiler_params=pltpu.CompilerParams(
            dimension_semantics=("arbitrary", "arbitrary")),
    )(adj1, adj2)


def _body(adj1_ref, adj2_ref, x_ref, y_ref, wl_ref, bl_ref, w4_ref, b4_ref,
          w5_ref, b5_ref, wmt_ref, bm_ref,
          xlr_ref, ylr_ref, xfin_ref, yfin_ref,
          xfiv_ref, xmlp_ref, yfiv_ref, ymlp_ref,
          lr_f32, lr_bf16):
    p = pl.program_id(0)
    i = pl.program_id(1)
    bf16 = jnp.bfloat16

    def layer1(adj_ref, feat_ref, lr_out_ref):
        ax = _dot(adj_ref[...], feat_ref[...])
        s = jax.nn.sigmoid(_dot(ax.astype(bf16), wl_ref[...]) + bl_ref[...])
        lane = jax.lax.broadcasted_iota(jnp.int32, (1, _SUMF), 1)
        sec_mask = jnp.logical_and(lane >= _F0, lane < _F0 + _F1)
        msec = jnp.sum(jnp.where(sec_mask, s, 0.0), axis=1,
                       keepdims=True) * (1.0 / _F1)
        lrb = s * jnp.where(lane < _F0 + _F1, 1.0, msec)
        lr_out_ref[...] = lrb
        lr_f32[pl.ds(i * _BMC, _BMC), :] = lrb
        lr_bf16[pl.ds(i * _BMC, _BMC), :] = lrb.astype(bf16)

    def layer2(adj_ref, final_ref, fiv_ref, mlp_ref):
        alr = _dot(adj_ref[...], lr_bf16[...])
        alrb = alr.astype(bf16)
        fou = _dot(alrb, w4_ref[...]) + b4_ref[...]
        fiv = _dot(alrb, w5_ref[...]) + b5_ref[...]
        m = _dot(fiv.astype(bf16), wmt_ref[...]) + bm_ref[...]
        m = jnp.where(m >= 0, m, 0.01 * m)
        f3 = (m + fou) * 0.5
        lrb = lr_f32[pl.ds(i * _BMC, _BMC), :]
        low = jnp.mean(lrb, axis=1, keepdims=True) * lrb + lrb
        final_ref[...] = jnp.concatenate([low, f3], axis=1)
        fiv_ref[...] = fiv
        mlp_ref[...] = m

    @pl.when(p == 0)
    def _():
        layer1(adj1_ref, x_ref, xlr_ref)

    @pl.when(p == 1)
    def _():
        layer2(adj1_ref, xfin_ref, xfiv_ref, xmlp_ref)

    @pl.when(p == 2)
    def _():
        layer1(adj2_ref, y_ref, ylr_ref)

    @pl.when(p == 3)
    def _():
        layer2(adj2_ref, yfin_ref, yfiv_ref, ymlp_ref)


def _const(shape):
    return pl.BlockSpec(shape, lambda p, i: tuple(0 for _ in shape))


def kernel(x, adj1, y, adj2, W1, b1, W2, b2, W3, b3, W4, b4, W5, b5, Wm, bm):
    f32 = jnp.float32
    bf16 = jnp.bfloat16
    wl = jnp.concatenate([W1, W2, W3], axis=1).astype(bf16)
    bl = jnp.concatenate([b1, b2, b3]).reshape(1, _SUMF)
    b4r = b4.reshape(1, _H4)
    b5r = b5.reshape(1, _H5)
    bmr = bm.reshape(1, _H4)
    xb = x.astype(bf16)
    yb = y.astype(bf16)
    w4b = W4.astype(bf16)
    w5b = W5.astype(bf16)
    wmtb = Wm.T.astype(bf16)

    adj1b, adj2b = _cast_pass(adj1, adj2)

    last = _NBC - 1

    def adj1_map(p, i):
        c = p // 2
        return (i * (1 - c) + last * c, 0)

    def adj2_map(p, i):
        c = p // 2
        return (i * c, 0)

    def xlr_map(p, i):
        a = (p + 3) // 4
        return (i * (1 - a) + last * a, 0)

    def xtail_map(p, i):
        a = (p + 3) // 4
        b = p // 2
        return (i * (a - b) + last * b, 0)

    def ylr_map(p, i):
        c = p // 2
        d = p // 3
        return (i * (c - d) + last * d, 0)

    def ytail_map(p, i):
        d = p // 3
        return (i * d, 0)

    x_lr, y_lr, x_final, y_final, x_fiv, x_mlp, y_fiv, y_mlp = pl.pallas_call(
        _body,
        grid=(4, _NBC),
        in_specs=[
            pl.BlockSpec((_BMC, _N), adj1_map),
            pl.BlockSpec((_BMC, _N), adj2_map),
            _const((_N, _NFEAT)),
            _const((_N, _NFEAT)),
            _const((_NFEAT, _SUMF)),
            _const((1, _SUMF)),
            _const((_SUMF, _H4)),
            _const((1, _H4)),
            _const((_SUMF, _H5)),
            _const((1, _H5)),
            _const((_H5, _H4)),
            _const((1, _H4)),
        ],
        out_specs=[
            pl.BlockSpec((_BMC, _SUMF), xlr_map),
            pl.BlockSpec((_BMC, _SUMF), ylr_map),
            pl.BlockSpec((_BMC, _SUMF + _H4), xtail_map),
            pl.BlockSpec((_BMC, _SUMF + _H4), ytail_map),
            pl.BlockSpec((_BMC, _H5), xtail_map),
            pl.BlockSpec((_BMC, _H4), xtail_map),
            pl.BlockSpec((_BMC, _H5), ytail_map),
            pl.BlockSpec((_BMC, _H4), ytail_map),
        ],
        out_shape=[
            jax.ShapeDtypeStruct((_N, _SUMF), f32),
            jax.ShapeDtypeStruct((_N, _SUMF), f32),
            jax.ShapeDtypeStruct((_N, _SUMF + _H4), f32),
            jax.ShapeDtypeStruct((_N, _SUMF + _H4), f32),
            jax.ShapeDtypeStruct((_N, _H5), f32),
            jax.ShapeDtypeStruct((_N, _H4), f32),
            jax.ShapeDtypeStruct((_N, _H5), f32),
            jax.ShapeDtypeStruct((_N, _H4), f32),
        ],
        scratch_shapes=[
            pltpu.VMEM((_N, _SUMF), f32),
            pltpu.VMEM((_N, _SUMF), jnp.bfloat16),
        ],
        compiler_params=pltpu.CompilerParams(
            dimension_semantics=("arbitrary", "arbitrary")),
    )(adj1b, adj2b, xb, yb, wl, bl, w4b, b4r, w5b, b5r, wmtb, bmr)
    return (x_lr, y_lr, x_final, y_final, x_fiv, x_mlp, y_fiv, y_mlp)


# speedup vs baseline: 1.1484x; 1.0861x over previous
import jax
import jax.numpy as jnp
from jax.experimental import pallas as pl
from jax.experimental.pallas import tpu as pltpu

_N = 4096
_NFEAT = 128
_F0, _F1, _F2 = 64, 68, 72
_SUMF = _F0 + _F1 + _F2
_H4 = _F0 * 2 + 4
_H5 = _F0 * 2
_BM = 512
_NB = _N // _BM


def _dot(a, b):
    return jnp.dot(a, b, preferred_element_type=jnp.float32)


def _layer1_body(adj1_ref, adj2_ref, x_ref, y_ref, wl_ref, bl_ref,
                 xlr_ref, ylr_ref, xlrb_ref, ylrb_ref, adj1b_ref, adj2b_ref):
    p = pl.program_id(0)
    bf16 = jnp.bfloat16

    def layer1(adj_ref, feat_ref, lr_ref, lrb_ref, adjb_ref):
        a = adj_ref[...]
        adjb_ref[...] = a.astype(bf16)
        ax = _dot(a, feat_ref[...])
        s = jax.nn.sigmoid(_dot(ax.astype(bf16), wl_ref[...]) + bl_ref[...])
        lane = jax.lax.broadcasted_iota(jnp.int32, (1, _SUMF), 1)
        sec_mask = jnp.logical_and(lane >= _F0, lane < _F0 + _F1)
        msec = jnp.sum(jnp.where(sec_mask, s, 0.0), axis=1,
                       keepdims=True) * (1.0 / _F1)
        lrb = s * jnp.where(lane < _F0 + _F1, 1.0, msec)
        lr_ref[...] = lrb
        lrb_ref[...] = lrb.astype(bf16)

    @pl.when(p == 0)
    def _():
        layer1(adj1_ref, x_ref, xlr_ref, xlrb_ref, adj1b_ref)

    @pl.when(p == 1)
    def _():
        layer1(adj2_ref, y_ref, ylr_ref, ylrb_ref, adj2b_ref)


def _layer2_body(adj1b_ref, adj2b_ref, xlr_ref, ylr_ref, xlrb_ref, ylrb_ref,
                 w4_ref, b4_ref, w5_ref, b5_ref, wmt_ref, bm_ref,
                 xfin_ref, yfin_ref, xfiv_ref, xmlp_ref, yfiv_ref, ymlp_ref):
    p = pl.program_id(0)
    i = pl.program_id(1)
    bf16 = jnp.bfloat16

    def layer2(adjb_ref, lr_ref, lrb_ref, final_ref, fiv_ref, mlp_ref):
        alr = _dot(adjb_ref[...], lrb_ref[...])
        alrb = alr.astype(bf16)
        fou = _dot(alrb, w4_ref[...]) + b4_ref[...]
        fiv = _dot(alrb, w5_ref[...]) + b5_ref[...]
        m = _dot(fiv.astype(bf16), wmt_ref[...]) + bm_ref[...]
        m = jnp.where(m >= 0, m, 0.01 * m)
        f3 = (m + fou) * 0.5
        lrb = lr_ref[pl.ds(i * _BM, _BM), :]
        low = jnp.mean(lrb, axis=1, keepdims=True) * lrb + lrb
        final_ref[...] = jnp.concatenate([low, f3], axis=1)
        fiv_ref[...] = fiv
        mlp_ref[...] = m

    @pl.when(p == 0)
    def _():
        layer2(adj1b_ref, xlr_ref, xlrb_ref, xfin_ref, xfiv_ref, xmlp_ref)

    @pl.when(p == 1)
    def _():
        layer2(adj2b_ref, ylr_ref, ylrb_ref, yfin_ref, yfiv_ref, ymlp_ref)


def _const(shape):
    return pl.BlockSpec(shape, lambda p, i: tuple(0 for _ in shape))


def kernel(x, adj1, y, adj2, W1, b1, W2, b2, W3, b3, W4, b4, W5, b5, Wm, bm):
    f32 = jnp.float32
    bf16 = jnp.bfloat16
    wl = jnp.concatenate([W1, W2, W3], axis=1).astype(bf16)
    bl = jnp.concatenate([b1, b2, b3]).reshape(1, _SUMF)
    b4r = b4.reshape(1, _H4)
    b5r = b5.reshape(1, _H5)
    bmr = bm.reshape(1, _H4)
    xb = x.astype(bf16)
    yb = y.astype(bf16)
    w4b = W4.astype(bf16)
    w5b = W5.astype(bf16)
    wmtb = Wm.T.astype(bf16)

    last = _NB - 1

    def s1_map(p, i):
        return (i * (1 - p) + last * p, 0)

    def s2_map(p, i):
        return (i * p, 0)

    xlr, ylr, xlrb, ylrb, adj1b, adj2b = pl.pallas_call(
        _layer1_body,
        grid=(2, _NB),
        in_specs=[
            pl.BlockSpec((_BM, _N), s1_map),
            pl.BlockSpec((_BM, _N), s2_map),
            _const((_N, _NFEAT)),
            _const((_N, _NFEAT)),
            _const((_NFEAT, _SUMF)),
            _const((1, _SUMF)),
        ],
        out_specs=[
            pl.BlockSpec((_BM, _SUMF), s1_map),
            pl.BlockSpec((_BM, _SUMF), s2_map),
            pl.BlockSpec((_BM, _SUMF), s1_map),
            pl.BlockSpec((_BM, _SUMF), s2_map),
            pl.BlockSpec((_BM, _N), s1_map),
            pl.BlockSpec((_BM, _N), s2_map),
        ],
        out_shape=[
            jax.ShapeDtypeStruct((_N, _SUMF), f32),
            jax.ShapeDtypeStruct((_N, _SUMF), f32),
            jax.ShapeDtypeStruct((_N, _SUMF), bf16),
            jax.ShapeDtypeStruct((_N, _SUMF), bf16),
            jax.ShapeDtypeStruct((_N, _N), bf16),
            jax.ShapeDtypeStruct((_N, _N), bf16),
        ],
        compiler_params=pltpu.CompilerParams(
            dimension_semantics=("arbitrary", "arbitrary")),
    )(adj1, adj2, xb, yb, wl, bl)

    xfin, yfin, xfiv, xmlp, yfiv, ymlp = pl.pallas_call(
        _layer2_body,
        grid=(2, _NB),
        in_specs=[
            pl.BlockSpec((_BM, _N), s1_map),
            pl.BlockSpec((_BM, _N), s2_map),
            _const((_N, _SUMF)),
            _const((_N, _SUMF)),
            _const((_N, _SUMF)),
            _const((_N, _SUMF)),
            _const((_SUMF, _H4)),
            _const((1, _H4)),
            _const((_SUMF, _H5)),
            _const((1, _H5)),
            _const((_H5, _H4)),
            _const((1, _H4)),
        ],
        out_specs=[
            pl.BlockSpec((_BM, _SUMF + _H4), s1_map),
            pl.BlockSpec((_BM, _SUMF + _H4), s2_map),
            pl.BlockSpec((_BM, _H5), s1_map),
            pl.BlockSpec((_BM, _H4), s1_map),
            pl.BlockSpec((_BM, _H5), s2_map),
            pl.BlockSpec((_BM, _H4), s2_map),
        ],
        out_shape=[
            jax.ShapeDtypeStruct((_N, _SUMF + _H4), f32),
            jax.ShapeDtypeStruct((_N, _SUMF + _H4), f32),
            jax.ShapeDtypeStruct((_N, _H5), f32),
            jax.ShapeDtypeStruct((_N, _H4), f32),
            jax.ShapeDtypeStruct((_N, _H5), f32),
            jax.ShapeDtypeStruct((_N, _H4), f32),
        ],
        compiler_params=pltpu.CompilerParams(
            dimension_semantics=("arbitrary", "arbitrary")),
    )(adj1b, adj2b, xlr, ylr, xlrb, ylrb, w4b, b4r, w5b, b5r, wmtb, bmr)

    return (xlr, ylr, xfin, yfin, xfiv, xmlp, yfiv, ymlp)


# speedup vs baseline: 1.2708x; 1.1066x over previous
import jax
import jax.numpy as jnp
from jax.experimental import pallas as pl
from jax.experimental.pallas import tpu as pltpu

_N = 4096
_NFEAT = 128
_F0, _F1, _F2 = 64, 68, 72
_SUMF = _F0 + _F1 + _F2
_H4 = _F0 * 2 + 4
_H5 = _F0 * 2
_BM = 512
_NB = _N // _BM


def _dot(a, b):
    return jnp.dot(a, b, preferred_element_type=jnp.float32)


def _body(adj1_ref, adj2_ref, x_ref, y_ref, wl_ref, bl_ref, w4_ref, b4_ref,
          w5_ref, b5_ref, wmt_ref, bm_ref,
          xlr_ref, ylr_ref, xfin_ref, yfin_ref,
          xfiv_ref, xmlp_ref, yfiv_ref, ymlp_ref,
          lr_f32, lr_bf16):
    p = pl.program_id(0)
    i = pl.program_id(1)
    bf16 = jnp.bfloat16

    def layer1(adj_ref, feat_ref, lr_out_ref):
        ax = _dot(adj_ref[...].astype(bf16), feat_ref[...])
        s = jax.nn.sigmoid(_dot(ax, wl_ref[...]) + bl_ref[...])
        fir = s[:, :_F0]
        sec = s[:, _F0:_F0 + _F1]
        thi = s[:, _F0 + _F1:]
        f2 = jnp.mean(sec, axis=1, keepdims=True) * thi
        lrb = jnp.concatenate([fir, sec, f2], axis=1)
        lr_out_ref[...] = lrb
        lr_f32[pl.ds(i * _BM, _BM), :] = lrb
        lr_bf16[pl.ds(i * _BM, _BM), :] = lrb.astype(bf16)

    def layer2(adj_ref, final_ref, fiv_ref, mlp_ref):
        alr = _dot(adj_ref[...].astype(bf16), lr_bf16[...])
        fou = _dot(alr, w4_ref[...]) + b4_ref[...]
        fiv = _dot(alr, w5_ref[...]) + b5_ref[...]
        m = _dot(fiv, wmt_ref[...]) + bm_ref[...]
        m = jnp.where(m >= 0, m, 0.01 * m)
        f3 = (m + fou) * 0.5
        lrb = lr_f32[pl.ds(i * _BM, _BM), :]
        low = jnp.mean(lrb, axis=1, keepdims=True) * lrb + lrb
        final_ref[...] = jnp.concatenate([low, f3], axis=1)
        fiv_ref[...] = fiv
        mlp_ref[...] = m

    @pl.when(p == 0)
    def _():
        layer1(adj1_ref, x_ref, xlr_ref)

    @pl.when(p == 1)
    def _():
        layer2(adj1_ref, xfin_ref, xfiv_ref, xmlp_ref)

    @pl.when(p == 2)
    def _():
        layer1(adj2_ref, y_ref, ylr_ref)

    @pl.when(p == 3)
    def _():
        layer2(adj2_ref, yfin_ref, yfiv_ref, ymlp_ref)


def _const(shape):
    return pl.BlockSpec(shape, lambda p, i: tuple(0 for _ in shape))


def kernel(x, adj1, y, adj2, W1, b1, W2, b2, W3, b3, W4, b4, W5, b5, Wm, bm):
    f32 = jnp.float32
    wl = jnp.concatenate([W1, W2, W3], axis=1)
    bl = jnp.concatenate([b1, b2, b3]).reshape(1, _SUMF)
    b4r = b4.reshape(1, _H4)
    b5r = b5.reshape(1, _H5)
    wmt = Wm.T
    bmr = bm.reshape(1, _H4)
    xb = x.astype(jnp.bfloat16)
    yb = y.astype(jnp.bfloat16)

    last = _NB - 1

    def adj1_map(p, i):
        c = p // 2
        return (i * (1 - c) + last * c, 0)

    def adj2_map(p, i):
        c = p // 2
        return (i * c, 0)

    def xlr_map(p, i):
        a = (p + 3) // 4
        return (i * (1 - a) + last * a, 0)

    def xtail_map(p, i):
        a = (p + 3) // 4
        b = p // 2
        return (i * (a - b) + last * b, 0)

    def ylr_map(p, i):
        c = p // 2
        d = p // 3
        return (i * (c - d) + last * d, 0)

    def ytail_map(p, i):
        d = p // 3
        return (i * d, 0)

    x_lr, y_lr, x_final, y_final, x_fiv, x_mlp, y_fiv, y_mlp = pl.pallas_call(
        _body,
        grid=(4, _NB),
        in_specs=[
            pl.BlockSpec((_BM, _N), adj1_map),
            pl.BlockSpec((_BM, _N), adj2_map),
            _const((_N, _NFEAT)),
            _const((_N, _NFEAT)),
            _const((_NFEAT, _SUMF)),
            _const((1, _SUMF)),
            _const((_SUMF, _H4)),
            _const((1, _H4)),
            _const((_SUMF, _H5)),
            _const((1, _H5)),
            _const((_H5, _H4)),
            _const((1, _H4)),
        ],
        out_specs=[
            pl.BlockSpec((_BM, _SUMF), xlr_map),
            pl.BlockSpec((_BM, _SUMF), ylr_map),
            pl.BlockSpec((_BM, _SUMF + _H4), xtail_map),
            pl.BlockSpec((_BM, _SUMF + _H4), ytail_map),
            pl.BlockSpec((_BM, _H5), xtail_map),
            pl.BlockSpec((_BM, _H4), xtail_map),
            pl.BlockSpec((_BM, _H5), ytail_map),
            pl.BlockSpec((_BM, _H4), ytail_map),
        ],
        out_shape=[
            jax.ShapeDtypeStruct((_N, _SUMF), f32),
            jax.ShapeDtypeStruct((_N, _SUMF), f32),
            jax.ShapeDtypeStruct((_N, _SUMF + _H4), f32),
            jax.ShapeDtypeStruct((_N, _SUMF + _H4), f32),
            jax.ShapeDtypeStruct((_N, _H5), f32),
            jax.ShapeDtypeStruct((_N, _H4), f32),
            jax.ShapeDtypeStruct((_N, _H5), f32),
            jax.ShapeDtypeStruct((_N, _H4), f32),
        ],
        scratch_shapes=[
            pltpu.VMEM((_N, _SUMF), f32),
            pltpu.VMEM((_N, _SUMF), jnp.bfloat16),
        ],
        compiler_params=pltpu.CompilerParams(
            dimension_semantics=("arbitrary", "arbitrary")),
    )(adj1, adj2, xb, yb, wl, bl, W4, b4r, W5, b5r, wmt, bmr)
    return (x_lr, y_lr, x_final, y_final, x_fiv, x_mlp, y_fiv, y_mlp)
